# baseline (speedup 1.0000x reference)
import functools

import jax
import jax.numpy as jnp
from jax import lax
from jax.experimental import pallas as pl
from jax.experimental.pallas import tpu as pltpu
from jax.experimental.pallas import tpu_sc as plsc

_B = 4096
_M = 4
_D = 64
_NGROUPS = 4096
_NUSERS = 100000

_NC = 2
_NS = 16
_NW = _NC * _NS
_PW = _B // _NW

_UBLK = (_NUSERS + 127) // 128
_GBLK = _NGROUPS // 128

_ME_ROWS = _B * _M + 128
_R_ROWS = _B + 128


def _sc_gather2(gi, ii, gm_flat, ut_p, it_p, gt_p):
    mesh = plsc.VectorSubcoreMesh(core_axis_name="c", subcore_axis_name="s")

    @functools.partial(
        pl.kernel,
        mesh=mesh,
        compiler_params=pltpu.CompilerParams(needs_layout_passes=False),
        out_type=(
            jax.ShapeDtypeStruct((_ME_ROWS, 128), jnp.float32),
            jax.ShapeDtypeStruct((_R_ROWS, 128), jnp.float32),
            jax.ShapeDtypeStruct((_R_ROWS, 128), jnp.float32),
        ),
        scratch_types=[
            pltpu.VMEM((512,), jnp.int32),
            pltpu.VMEM((512,), jnp.int32),
            pltpu.VMEM((128,), jnp.int32),
            pltpu.VMEM((128,), jnp.int32),
            pltpu.VMEM((_NGROUPS * _M,), jnp.int32),
            pltpu.VMEM((128, 128), jnp.float32),
            pltpu.VMEM((128, 128), jnp.float32),
            pltpu.SemaphoreType.DMA,
            pltpu.SemaphoreType.DMA,
        ],
    )
    def k(gi_hbm, ii_hbm, gm_hbm, ut_hbm, it_hbm, gt_hbm,
          me_out, ir_out, gr_out,
          gid_v, uidx_v, ii_v, gi_v, gm_v, rows_a, rows_b, sem_a, sem_b):
        w = lax.axis_index("s") * _NC + lax.axis_index("c")
        m_own = w // 8
        bbase = (w % 8) * 512
        pltpu.sync_copy(gi_hbm.at[pl.ds(bbase, 512)], gid_v)
        pltpu.sync_copy(ii_hbm.at[pl.ds(w * 128, 128)], ii_v)
        pltpu.sync_copy(gi_hbm.at[pl.ds(w * 128, 128)], gi_v)
        pltpu.sync_copy(gm_hbm, gm_v)
        for i in range(32):
            g16 = gid_v[pl.ds(i * 16, 16)]
            uidx_v[pl.ds(i * 16, 16)] = plsc.load_gather(
                gm_v, [g16 * _M + m_own])

        chunks = [(ut_hbm.at[uidx_v.at[pl.ds(c * 128, 128)]],
                   me_out.at[pl.ds(m_own * _B + bbase + c * 128, 128)])
                  for c in range(4)]
        chunks.append((it_hbm.at[ii_v], ir_out.at[pl.ds(w * 128, 128)]))
        chunks.append((gt_hbm.at[gi_v], gr_out.at[pl.ds(w * 128, 128)]))

        bufs = [rows_a, rows_b]
        sems = [sem_a, sem_b]
        cps = [None, None]
        cps[0] = pltpu.async_copy(chunks[0][0], bufs[0], sems[0])
        for i in range(len(chunks)):
            p = i % 2
            if i + 1 < len(chunks):
                cps[1 - p] = pltpu.async_copy(
                    chunks[i + 1][0], bufs[1 - p], sems[1 - p])
            cps[p].wait()
            pltpu.sync_copy(bufs[p], chunks[i][1])

    return k(gi, ii, gm_flat, ut_p, it_p, gt_p)


_BLK = 512


def _tc_body(me0_ref, me1_ref, me2_ref, me3_ref, it_ref, gr_ref,
             w1u_ref, w1i_ref, b1_ref, w2_ref, b2_ref,
             wc_ref, bc_ref, wp1_ref, bp1_ref, wp2_ref, bp2_ref,
             y_ref, aw_ref, ty_ref):
    me = [me0_ref[...][:, :_D], me1_ref[...][:, :_D],
          me2_ref[...][:, :_D], me3_ref[...][:, :_D]]
    item = it_ref[...][:, :_D]
    grp = gr_ref[...][:, :_D]
    w1u = w1u_ref[...]
    b1 = b1_ref[...]
    w2 = w2_ref[...]

    t = jnp.dot(item, w1i_ref[...]) + b1
    cols = []
    for m in range(_M):
        h = jnp.maximum(jnp.dot(me[m], w1u) + t, 0.0)
        cols.append(jnp.dot(h, w2))
    logits = jnp.concatenate(cols, axis=1) + b2_ref[...]

    mx = jnp.max(logits, axis=1, keepdims=True)
    e = jnp.exp(logits - mx)
    aw = e / jnp.sum(e, axis=1, keepdims=True)

    mw = jnp.max(aw, axis=1, keepdims=True)
    iota4 = lax.broadcasted_iota(jnp.int32, (_BLK, _M), 1).astype(jnp.float32)
    idx = jnp.min(jnp.where(aw >= mw, iota4, float(_M)), axis=1, keepdims=True)
    oh = (iota4 == idx).astype(jnp.float32)

    wc = wc_ref[...]
    bc = bc_ref[...]
    diff = aw * (wc[:, 1:2] - wc[:, 0:1]) + (bc[:, 1:2] - bc[:, 0:1])
    pred = (diff > 0.0).astype(jnp.float32)
    ptype = jnp.sum(oh * pred, axis=1, keepdims=True)

    wsel = jnp.where(ptype == 1.0, oh, aw)
    g = wsel[:, 0:1] * me[0]
    for m in range(1, _M):
        g = g + wsel[:, m:m + 1] * me[m]

    ge = g + grp
    el = ge * item
    new = jnp.concatenate([el, ge, item], axis=1)
    p = jnp.maximum(jnp.dot(new, wp1_ref[...]) + bp1_ref[...], 0.0)
    y = jax.nn.sigmoid(jnp.dot(p, wp2_ref[...]) + bp2_ref[...])

    y_ref[...] = y
    aw_ref[...] = aw
    ty_ref[...] = ptype


def _tc_dense(me_p, ir_p, gr_p, w1u, w1i, b1, w2, b2, wc, bc,
              wp1, bp1, wp2, bp2):
    grid = _B // _BLK
    full = lambda a: pl.BlockSpec(a.shape, lambda i: (0,) * a.ndim)
    me_spec = lambda m: pl.BlockSpec(
        (_BLK, 128), lambda i, m=m: (m * grid + i, 0))
    return pl.pallas_call(
        _tc_body,
        grid=(grid,),
        in_specs=[
            me_spec(0), me_spec(1), me_spec(2), me_spec(3),
            pl.BlockSpec((_BLK, 128), lambda i: (i, 0)),
            pl.BlockSpec((_BLK, 128), lambda i: (i, 0)),
            full(w1u), full(w1i), full(b1), full(w2), full(b2),
            full(wc), full(bc), full(wp1), full(bp1), full(wp2), full(bp2),
        ],
        out_specs=[
            pl.BlockSpec((_BLK, 1), lambda i: (i, 0)),
            pl.BlockSpec((_BLK, _M), lambda i: (i, 0)),
            pl.BlockSpec((_BLK, 1), lambda i: (i, 0)),
        ],
        out_shape=[
            jax.ShapeDtypeStruct((_B, 1), jnp.float32),
            jax.ShapeDtypeStruct((_B, _M), jnp.float32),
            jax.ShapeDtypeStruct((_B, 1), jnp.float32),
        ],
    )(me_p, me_p, me_p, me_p, ir_p, gr_p, w1u, w1i, b1, w2, b2, wc, bc,
      wp1, bp1, wp2, bp2)


def kernel(group_inputs, item_inputs, group_members, user_table, item_table,
           group_table, W1, b1, W2, b2, Wc, bc, Wp1, bp1, Wp2, bp2):
    pad = lambda t: jnp.pad(t, ((0, 0), (0, 64)))
    me_p, ir_p, gr_p = _sc_gather2(
        group_inputs, item_inputs, group_members.reshape(-1),
        pad(user_table), pad(item_table), pad(group_table))

    y, aw, ty = _tc_dense(
        me_p, ir_p, gr_p,
        W1[:_D], W1[_D:], b1.reshape(1, 16), W2, b2.reshape(1, 1),
        Wc, bc.reshape(1, 2), Wp1, bp1.reshape(1, 8), Wp2, bp2.reshape(1, 1))
    return y, aw, ty.reshape(_B)


# speedup vs baseline: 1.1859x; 1.1859x over previous
import functools

import jax
import jax.numpy as jnp
from jax import lax
from jax.experimental import pallas as pl
from jax.experimental.pallas import tpu as pltpu
from jax.experimental.pallas import tpu_sc as plsc

_B = 4096
_M = 4
_D = 64
_NGROUPS = 4096
_NUSERS = 100000

_NC = 2
_NS = 16
_NW = _NC * _NS
_PW = _B // _NW

_UBLK = (_NUSERS + 127) // 128
_GBLK = _NGROUPS // 128

_ME_ROWS = _B * _M + 128
_R_ROWS = _B + 128


def _sc_gather2(gi, ii, gm_flat, ut_p, it_p, gt_p):
    mesh = plsc.VectorSubcoreMesh(core_axis_name="c", subcore_axis_name="s")

    @functools.partial(
        pl.kernel,
        mesh=mesh,
        compiler_params=pltpu.CompilerParams(needs_layout_passes=False),
        out_type=(
            jax.ShapeDtypeStruct((_ME_ROWS, 128), jnp.float32),
            jax.ShapeDtypeStruct((_R_ROWS, 128), jnp.float32),
            jax.ShapeDtypeStruct((_R_ROWS, 128), jnp.float32),
        ),
        scratch_types=[
            pltpu.VMEM((512,), jnp.int32),
            pltpu.VMEM((512,), jnp.int32),
            pltpu.VMEM((128,), jnp.int32),
            pltpu.VMEM((128,), jnp.int32),
            pltpu.VMEM((_NGROUPS * _M,), jnp.int32),
            pltpu.VMEM((128, 128), jnp.float32),
            pltpu.VMEM((128, 128), jnp.float32),
            pltpu.SemaphoreType.DMA,
            pltpu.SemaphoreType.DMA,
        ],
    )
    def k(gi_hbm, ii_hbm, gm_hbm, ut_hbm, it_hbm, gt_hbm,
          me_out, ir_out, gr_out,
          gid_v, uidx_v, ii_v, gi_v, gm_v, rows_a, rows_b, sem_a, sem_b):
        w = lax.axis_index("s") * _NC + lax.axis_index("c")
        m_own = w // 8
        bbase = (w % 8) * 512
        pltpu.sync_copy(gi_hbm.at[pl.ds(bbase, 512)], gid_v)
        pltpu.sync_copy(ii_hbm.at[pl.ds(w * 128, 128)], ii_v)
        pltpu.sync_copy(gi_hbm.at[pl.ds(w * 128, 128)], gi_v)
        pltpu.sync_copy(gm_hbm, gm_v)
        for i in range(32):
            g16 = gid_v[pl.ds(i * 16, 16)]
            uidx_v[pl.ds(i * 16, 16)] = plsc.load_gather(
                gm_v, [g16 * _M + m_own])

        chunks = [(ut_hbm.at[uidx_v.at[pl.ds(c * 128, 128)]],
                   me_out.at[pl.ds(m_own * _B + bbase + c * 128, 128)])
                  for c in range(4)]
        chunks.append((it_hbm.at[ii_v], ir_out.at[pl.ds(w * 128, 128)]))
        chunks.append((gt_hbm.at[gi_v], gr_out.at[pl.ds(w * 128, 128)]))

        bufs = [rows_a, rows_b]
        sems = [sem_a, sem_b]
        cps = [None, None]
        cps[0] = pltpu.async_copy(chunks[0][0], bufs[0], sems[0])
        for i in range(len(chunks)):
            p = i % 2
            if i + 1 < len(chunks):
                cps[1 - p] = pltpu.async_copy(
                    chunks[i + 1][0], bufs[1 - p], sems[1 - p])
            cps[p].wait()
            pltpu.sync_copy(bufs[p], chunks[i][1])

    return k(gi, ii, gm_flat, ut_p, it_p, gt_p)


def _tc_pad_body(xt_ref, eye_ref, out_ref):
    xt = xt_ref[...]
    ident = eye_ref[...]
    rows = lax.dot_general(xt, ident, (((0,), (0,)), ((), ())))
    out_ref[:, pl.ds(0, _D)] = rows


def _tc_pad(table_t, n_rows, cb):
    grid = (n_rows + cb - 1) // cb
    eye = jnp.eye(_D, dtype=jnp.float32)
    return pl.pallas_call(
        _tc_pad_body,
        grid=(grid,),
        in_specs=[
            pl.BlockSpec((_D, cb), lambda i: (0, i)),
            pl.BlockSpec((_D, _D), lambda i: (0, 0)),
        ],
        out_specs=pl.BlockSpec((cb, 128), lambda i: (i, 0)),
        out_shape=jax.ShapeDtypeStruct((n_rows, 128), jnp.float32),
    )(table_t, eye)


_BLK = 512


def _tc_body(me0_ref, me1_ref, me2_ref, me3_ref, it_ref, gr_ref,
             w1u_ref, w1i_ref, b1_ref, w2_ref, b2_ref,
             wc_ref, bc_ref, wp1_ref, bp1_ref, wp2_ref, bp2_ref,
             y_ref, aw_ref, ty_ref):
    me = [me0_ref[...][:, :_D], me1_ref[...][:, :_D],
          me2_ref[...][:, :_D], me3_ref[...][:, :_D]]
    item = it_ref[...][:, :_D]
    grp = gr_ref[...][:, :_D]
    w1u = w1u_ref[...]
    b1 = b1_ref[...]
    w2 = w2_ref[...]

    t = jnp.dot(item, w1i_ref[...]) + b1
    cols = []
    for m in range(_M):
        h = jnp.maximum(jnp.dot(me[m], w1u) + t, 0.0)
        cols.append(jnp.dot(h, w2))
    logits = jnp.concatenate(cols, axis=1) + b2_ref[...]

    mx = jnp.max(logits, axis=1, keepdims=True)
    e = jnp.exp(logits - mx)
    aw = e / jnp.sum(e, axis=1, keepdims=True)

    mw = jnp.max(aw, axis=1, keepdims=True)
    iota4 = lax.broadcasted_iota(jnp.int32, (_BLK, _M), 1).astype(jnp.float32)
    idx = jnp.min(jnp.where(aw >= mw, iota4, float(_M)), axis=1, keepdims=True)
    oh = (iota4 == idx).astype(jnp.float32)

    wc = wc_ref[...]
    bc = bc_ref[...]
    diff = aw * (wc[:, 1:2] - wc[:, 0:1]) + (bc[:, 1:2] - bc[:, 0:1])
    pred = (diff > 0.0).astype(jnp.float32)
    ptype = jnp.sum(oh * pred, axis=1, keepdims=True)

    wsel = jnp.where(ptype == 1.0, oh, aw)
    g = wsel[:, 0:1] * me[0]
    for m in range(1, _M):
        g = g + wsel[:, m:m + 1] * me[m]

    ge = g + grp
    el = ge * item
    new = jnp.concatenate([el, ge, item], axis=1)
    p = jnp.maximum(jnp.dot(new, wp1_ref[...]) + bp1_ref[...], 0.0)
    y = jax.nn.sigmoid(jnp.dot(p, wp2_ref[...]) + bp2_ref[...])

    y_ref[...] = y
    aw_ref[...] = aw
    ty_ref[...] = ptype


def _tc_dense(me_p, ir_p, gr_p, w1u, w1i, b1, w2, b2, wc, bc,
              wp1, bp1, wp2, bp2):
    grid = _B // _BLK
    full = lambda a: pl.BlockSpec(a.shape, lambda i: (0,) * a.ndim)
    me_spec = lambda m: pl.BlockSpec(
        (_BLK, 128), lambda i, m=m: (m * grid + i, 0))
    return pl.pallas_call(
        _tc_body,
        grid=(grid,),
        in_specs=[
            me_spec(0), me_spec(1), me_spec(2), me_spec(3),
            pl.BlockSpec((_BLK, 128), lambda i: (i, 0)),
            pl.BlockSpec((_BLK, 128), lambda i: (i, 0)),
            full(w1u), full(w1i), full(b1), full(w2), full(b2),
            full(wc), full(bc), full(wp1), full(bp1), full(wp2), full(bp2),
        ],
        out_specs=[
            pl.BlockSpec((_BLK, 1), lambda i: (i, 0)),
            pl.BlockSpec((_BLK, _M), lambda i: (i, 0)),
            pl.BlockSpec((_BLK, 1), lambda i: (i, 0)),
        ],
        out_shape=[
            jax.ShapeDtypeStruct((_B, 1), jnp.float32),
            jax.ShapeDtypeStruct((_B, _M), jnp.float32),
            jax.ShapeDtypeStruct((_B, 1), jnp.float32),
        ],
    )(me_p, me_p, me_p, me_p, ir_p, gr_p, w1u, w1i, b1, w2, b2, wc, bc,
      wp1, bp1, wp2, bp2)


def kernel(group_inputs, item_inputs, group_members, user_table, item_table,
           group_table, W1, b1, W2, b2, Wc, bc, Wp1, bp1, Wp2, bp2):
    me_p, ir_p, gr_p = _sc_gather2(
        group_inputs, item_inputs, group_members.reshape(-1),
        _tc_pad(user_table.T, _NUSERS, 4096),
        _tc_pad(item_table.T, _NUSERS, 4096),
        _tc_pad(group_table.T, _NGROUPS, 4096))

    y, aw, ty = _tc_dense(
        me_p, ir_p, gr_p,
        W1[:_D], W1[_D:], b1.reshape(1, 16), W2, b2.reshape(1, 1),
        Wc, bc.reshape(1, 2), Wp1, bp1.reshape(1, 8), Wp2, bp2.reshape(1, 1))
    return y, aw, ty.reshape(_B)
